# baseline (speedup 1.0000x reference)
import functools

import jax
import jax.numpy as jnp
from jax.experimental import pallas as pl
from jax.experimental.pallas import tpu as pltpu

_HIGH = jax.lax.Precision.HIGHEST


def _kern(scr_ref, fmap_ref, out_ref, cnt_ref, bval_ref, bfeat_ref, *,
          nhb, hb_sz, wf):
    hb = pl.program_id(1)
    scr = scr_ref[0]
    i_q = scr.shape[0]
    hb_n = hb_sz
    scr4 = scr.reshape(i_q, hb_n, 4, 4 * wf)
    t = scr4[:, :, 1, :] + scr4[:, :, 2, :]
    t2 = t.reshape(i_q * hb_n, 4 * wf)
    r_io = jax.lax.broadcasted_iota(jnp.int32, (4 * wf, wf), 0)
    c_io = jax.lax.broadcasted_iota(jnp.int32, (4 * wf, wf), 1)
    wm = jnp.logical_or(r_io == 4 * c_io + 1,
                        r_io == 4 * c_io + 2).astype(jnp.float32)
    s4 = jax.lax.dot_general(t2, wm, (((1,), (0,)), ((), ())),
                             precision=_HIGH)
    sflat = s4.reshape(i_q, hb_n * wf)
    mask = (sflat > 2.0).astype(jnp.float32)

    fm = fmap_ref[0].reshape(fmap_ref.shape[1], hb_n * wf)
    partial = jax.lax.dot_general(mask, fm, (((1,), (1,)), ((), ())),
                                  precision=_HIGH)
    cnt_blk = jnp.sum(mask, axis=1, keepdims=True)

    bmax = jnp.max(sflat, axis=1, keepdims=True)
    gidx = (jax.lax.broadcasted_iota(jnp.int32, (i_q, hb_n * wf), 1)
            + hb * (hb_n * wf))
    cand = jnp.where(sflat == bmax, gidx, jnp.int32(2147483647))
    fidx = jnp.min(cand, axis=1, keepdims=True)
    onehot = (gidx == fidx).astype(jnp.float32)
    bfeat_blk = jax.lax.dot_general(onehot, fm, (((1,), (1,)), ((), ())),
                                    precision=_HIGH)

    @pl.when(hb == 0)
    def _init():
        out_ref[0] = partial
        cnt_ref[...] = cnt_blk
        bval_ref[...] = bmax
        bfeat_ref[...] = bfeat_blk

    @pl.when(hb != 0)
    def _acc():
        out_ref[0] += partial
        cnt_ref[...] += cnt_blk
        upd = bmax > bval_ref[...]
        bval_ref[...] = jnp.where(upd, bmax, bval_ref[...])
        bfeat_ref[...] = jnp.where(upd, bfeat_blk, bfeat_ref[...])

    @pl.when(hb == nhb - 1)
    def _fin():
        cnt = cnt_ref[...]
        out_ref[0] = jnp.where(cnt > 0.0, out_ref[0] / cnt, bfeat_ref[...])


def kernel(features, scribbles):
    fmap = features[-1]
    b, c, hf, wf = fmap.shape
    i_q = scribbles.shape[1]
    assert scribbles.shape[2] == 4 * hf and scribbles.shape[3] == 4 * wf

    hb_sz = 16
    nhb = hf // hb_sz
    out = pl.pallas_call(
        functools.partial(_kern, nhb=nhb, hb_sz=hb_sz, wf=wf),
        grid=(b, nhb),
        in_specs=[
            pl.BlockSpec((1, i_q, 4 * hb_sz, 4 * wf),
                         lambda bb, hh: (bb, 0, hh, 0)),
            pl.BlockSpec((1, c, hb_sz, wf),
                         lambda bb, hh: (bb, 0, hh, 0)),
        ],
        out_specs=pl.BlockSpec((1, i_q, c), lambda bb, hh: (bb, 0, 0)),
        out_shape=jax.ShapeDtypeStruct((b, i_q, c), jnp.float32),
        scratch_shapes=[
            pltpu.VMEM((i_q, 1), jnp.float32),
            pltpu.VMEM((i_q, 1), jnp.float32),
            pltpu.VMEM((i_q, c), jnp.float32),
        ],
    )(scribbles, fmap)
    return out
